# Initial kernel scaffold; baseline (speedup 1.0000x reference)
import jax
import jax.numpy as jnp
from jax.experimental import pallas as pl


def kernel(x, edge_index, W1, b1, W2, b2, W3, b3, W4, b4, Wl, bl, Wo, bo):
    raise NotImplementedError("write your pallas kernel here")


# speedup vs baseline: 50.1303x; 50.1303x over previous
import functools

import jax
import jax.numpy as jnp
from jax import lax
from jax.experimental import pallas as pl
from jax.experimental.pallas import tpu as pltpu
from jax.experimental.pallas import tpu_sc as plsc

N_NODES = 512
N_EDGES = 262144
H = 32
NC = 2
NS = 16
NW = NC * NS
EPT = N_EDGES // NW
CHUNK = 128
NCHUNK = EPT // CHUNK
SEG = N_EDGES // NS
BD = 64


def _codes_body(e_ref, out_ref):
    e = e_ref[...]
    out_ref[...] = e[1] * N_NODES + e[0]


def _compute_codes(edge_index):
    e3 = edge_index.reshape(2, N_EDGES // 128, 128)
    codes = pl.pallas_call(
        _codes_body,
        out_shape=jax.ShapeDtypeStruct((N_EDGES // 128, 128), jnp.int32),
    )(e3)
    return codes.reshape(NW, NCHUNK, CHUNK)


def _sc_mask_body(codes_hbm, out_hbm, idx_v, ones_v, zeros_v, mask_sh):
    cid = lax.axis_index("c")
    sid = lax.axis_index("s")
    wid = cid * NS + sid

    def _fill_ones(i, c):
        ones_v[pl.ds(i * 16, 16)] = jnp.ones((16,), jnp.float32)
        return c

    lax.fori_loop(0, CHUNK // 16, _fill_ones, 0)

    def _fill_zeros(i, c):
        zeros_v[pl.ds(i * 16, 16)] = jnp.zeros((16,), jnp.float32)
        return c

    lax.fori_loop(0, SEG // 16, _fill_zeros, 0)

    pltpu.sync_copy(zeros_v, mask_sh.at[pl.ds(sid * SEG, SEG)])
    pltpu.sync_copy(codes_hbm.at[wid], idx_v)
    plsc.subcore_barrier()

    def _scatter(j, c):
        pltpu.sync_copy(ones_v, mask_sh.at[idx_v.at[j]], add=True)
        return c

    lax.fori_loop(0, NCHUNK, _scatter, 0)
    plsc.subcore_barrier()
    pltpu.sync_copy(mask_sh.at[pl.ds(sid * SEG, SEG)],
                    out_hbm.at[cid, pl.ds(sid * SEG, SEG)])


def _sc_mask(codes):
    mesh = plsc.VectorSubcoreMesh(core_axis_name="c", subcore_axis_name="s")
    return pl.kernel(
        _sc_mask_body,
        out_type=jax.ShapeDtypeStruct((NC, N_EDGES), jnp.float32),
        mesh=mesh,
        scratch_types=[
            pltpu.VMEM((NCHUNK, CHUNK), jnp.int32),
            pltpu.VMEM((CHUNK,), jnp.float32),
            pltpu.VMEM((SEG,), jnp.float32),
            pltpu.VMEM_SHARED((N_EDGES,), jnp.float32),
        ],
    )(codes)


def _layer_body(x_ref, wa_ref, ba_ref, wb_ref, bb_ref, cnt_ref, out_ref,
                a_s, bt_s):
    i = pl.program_id(0)
    d_in = x_ref.shape[1]

    @pl.when(i == 0)
    def _():
        x = x_ref[...]
        wa = wa_ref[...]
        wtop = wa[:d_in]
        wbot = wa[d_in:]
        a_s[...] = (jnp.dot(x, wtop - wbot, preferred_element_type=jnp.float32)
                    + ba_ref[...])
        bt_s[...] = jnp.dot(x, wbot, preferred_element_type=jnp.float32).T

    a_t = a_s[pl.ds(i * BD, BD), :].T
    pre = jnp.maximum(a_t[:, :, None] + bt_s[...][:, None, :], 0.0)
    pre2 = pre.reshape(H, BD * N_NODES)
    r2 = jnp.dot(wb_ref[...].T, pre2, preferred_element_type=jnp.float32)
    r = r2.reshape(H, BD, N_NODES)
    cnt = cnt_ref[...]
    present = (cnt[0] + cnt[1]) > 0.0
    r = jnp.where(present[None], r, -jnp.inf)
    m = jnp.max(r, axis=2)
    out_ref[...] = jnp.maximum(m.T + bb_ref[...], 0.0)


def _layer(x, wa, ba, wb, bb, cnt3):
    d_in = x.shape[1]
    return pl.pallas_call(
        _layer_body,
        grid=(N_NODES // BD,),
        in_specs=[
            pl.BlockSpec((N_NODES, d_in), lambda i: (0, 0)),
            pl.BlockSpec((2 * d_in, H), lambda i: (0, 0)),
            pl.BlockSpec((1, H), lambda i: (0, 0)),
            pl.BlockSpec((H, H), lambda i: (0, 0)),
            pl.BlockSpec((1, H), lambda i: (0, 0)),
            pl.BlockSpec((NC, BD, N_NODES), lambda i: (0, i, 0)),
        ],
        out_specs=pl.BlockSpec((BD, H), lambda i: (i, 0)),
        out_shape=jax.ShapeDtypeStruct((N_NODES, H), jnp.float32),
        scratch_shapes=[
            pltpu.VMEM((N_NODES, H), jnp.float32),
            pltpu.VMEM((H, N_NODES), jnp.float32),
        ],
    )(x, wa, ba.reshape(1, H), wb, bb.reshape(1, H), cnt3)


def _head_body(v_ref, wl_ref, bl_ref, wo_ref, bo_ref, out_ref):
    v = v_ref[...]
    z1 = jnp.maximum(
        jnp.dot(v, wl_ref[...], preferred_element_type=jnp.float32)
        + bl_ref[...], 0.0)
    z2 = jnp.maximum(
        jnp.dot(z1, wo_ref[...], preferred_element_type=jnp.float32)
        + bo_ref[...], 0.0)
    mx = jnp.max(z2, axis=1, keepdims=True)
    e = jnp.exp(z2 - mx)
    out_ref[...] = e / jnp.sum(e, axis=1, keepdims=True)


def _head(v, wl, bl, wo, bo):
    return pl.pallas_call(
        _head_body,
        out_shape=jax.ShapeDtypeStruct((1, N_NODES), jnp.float32),
    )(v, wl, bl.reshape(1, -1), wo, bo.reshape(1, -1))


def kernel(x, edge_index, W1, b1, W2, b2, W3, b3, W4, b4, Wl, bl, Wo, bo):
    codes = _compute_codes(edge_index)
    cnt = _sc_mask(codes)
    cnt3 = cnt.reshape(NC, N_NODES, N_NODES)
    h1 = _layer(x, W1, b1, W2, b2, cnt3)
    h2 = _layer(h1, W3, b3, W4, b4, cnt3)
    v = h2.reshape(1, N_NODES * H)
    out = _head(v, Wl, bl, Wo, bo)
    return out.reshape(N_NODES)
